# baseline (speedup 1.0000x reference)
import functools

import jax
import jax.numpy as jnp
from jax.experimental import pallas as pl
from jax.experimental.pallas import tpu as pltpu
from jax.experimental.pallas import tpu_sc as plsc

S, D, H, OUTD = 2048, 768, 3072, 768
G, EG = 4, 4
E = G * EG
BT = 256
NBCAP = (2 * S) // BT + E
PCAP = NBCAP * BT

_BF = jnp.bfloat16
_HI = jax.lax.Precision.HIGHEST


def _router_body(x_ref, wgT_ref, bg_ref, werT_ref, ber_ref,
                 ppos_ref, w_ref, be_ref, nb_ref):
    xb = x_ref[...]
    gl = jax.lax.dot_general(wgT_ref[...], xb, (((1,), (1,)), ((), ())),
                             preferred_element_type=jnp.float32) \
        + bg_ref[...]
    ridx = jax.lax.broadcasted_iota(jnp.int32, (G, S), 0)
    big = jnp.int32(G + 1)

    m = jnp.max(gl, axis=0, keepdims=True)
    egl = jnp.exp(gl - m)
    gp = egl / jnp.sum(egl, axis=0, keepdims=True)
    v1 = jnp.max(gp, axis=0, keepdims=True)
    i1 = jnp.min(jnp.where(gp == v1, ridx, big), axis=0, keepdims=True)
    gp2 = jnp.where(ridx == i1, -1.0, gp)
    v2 = jnp.max(gp2, axis=0, keepdims=True)
    i2 = jnp.min(jnp.where(gp2 == v2, ridx, big), axis=0, keepdims=True)

    ew = []
    ei = []
    for g in range(G):
        el = jax.lax.dot_general(werT_ref[g], xb, (((1,), (1,)), ((), ())),
                                 preferred_element_type=jnp.float32) \
            + ber_ref[g]
        mg = jnp.max(el, axis=0, keepdims=True)
        ei.append(jnp.min(jnp.where(el == mg, ridx, big), axis=0,
                          keepdims=True))
        ew.append(1.0 / jnp.sum(jnp.exp(el - mg), axis=0, keepdims=True))

    eids, ws = [], []
    for gsel, gwk in ((i1, v1), (i2, v2)):
        ew_sel = jnp.zeros((1, S), jnp.float32)
        ei_sel = jnp.zeros((1, S), jnp.int32)
        for g in range(G):
            hit = gsel == g
            ew_sel = jnp.where(hit, ew[g], ew_sel)
            ei_sel = jnp.where(hit, ei[g], ei_sel)
        eids.append(gsel * EG + ei_sel)
        ws.append(gwk * ew_sel)
    w_ref[...] = jnp.concatenate(ws, axis=0)

    e16 = jax.lax.broadcasted_iota(jnp.int32, (E, S), 0)
    o0 = (e16 == eids[0]).astype(jnp.float32)
    o1 = (e16 == eids[1]).astype(jnp.float32)
    t0 = jnp.sum(o0, axis=1, keepdims=True)
    t1 = jnp.sum(o1, axis=1, keepdims=True)
    cols = jax.lax.broadcasted_iota(jnp.int32, (S, S), 1)
    rows = jax.lax.broadcasted_iota(jnp.int32, (S, S), 0)
    stu = (rows < cols).astype(_BF)
    ob = jnp.concatenate([o0, o1], axis=0).astype(_BF)
    rex = jax.lax.dot_general(ob, stu, (((1,), (0,)), ((), ())),
                              preferred_element_type=jnp.float32)
    r0 = rex[0:E]
    r1 = rex[E:2 * E] + t0
    counts = t0 + t1
    pc = jnp.floor((counts + (BT - 1)) * (1.0 / BT)) * BT
    lt = (jax.lax.broadcasted_iota(jnp.int32, (E, E), 1)
          < jax.lax.broadcasted_iota(jnp.int32, (E, E), 0)).astype(jnp.float32)
    pstart = jax.lax.dot_general(lt, pc, (((1,), (0,)), ((), ())),
                                 preferred_element_type=jnp.float32,
                                 precision=_HI)
    p0 = jnp.sum(o0 * (pstart + r0), axis=0, keepdims=True)
    p1 = jnp.sum(o1 * (pstart + r1), axis=0, keepdims=True)
    ppos_ref[...] = jnp.concatenate([p0, p1], axis=0).astype(jnp.int32)
    bcols = jax.lax.broadcasted_iota(jnp.int32, (E, NBCAP), 1).astype(
        jnp.float32)
    pstart_blk = pstart * (1.0 / BT)
    be_ref[...] = (jnp.sum((pstart_blk <= bcols).astype(jnp.float32), axis=0,
                           keepdims=True) - 1.0).astype(jnp.int32)
    nb_ref[...] = (jnp.sum(pc, axis=0, keepdims=True) * (1.0 / BT)
                   ).astype(jnp.int32)


def _route(x_bf, Wg, bg, Wer, ber):
    wgT = Wg.T.astype(_BF)
    bg2 = bg.reshape(G, 1)
    werT = Wer.transpose(0, 2, 1).astype(_BF)
    ber3 = ber.reshape(G, EG, 1)
    return pl.pallas_call(
        _router_body,
        out_shape=(jax.ShapeDtypeStruct((2, S), jnp.int32),
                   jax.ShapeDtypeStruct((2, S), jnp.float32),
                   jax.ShapeDtypeStruct((1, NBCAP), jnp.int32),
                   jax.ShapeDtypeStruct((1, 1), jnp.int32)),
    )(x_bf, wgT, bg2, werT, ber3)


def _gather_scatter(table, src_idx, dst_idx):
    n = src_idx.shape[0]
    dm = table.shape[1]
    win = 128
    si = src_idx.reshape(1, n)
    di = dst_idx.reshape(1, n)
    mesh = plsc.VectorSubcoreMesh(core_axis_name="c", subcore_axis_name="s")

    @functools.partial(pl.kernel,
                       out_type=jax.ShapeDtypeStruct((n, dm), table.dtype),
                       mesh=mesh,
                       scratch_types=[pltpu.VMEM((win, dm), table.dtype)])
    def k(x_hbm, si_hbm, di_hbm, o_hbm, rows_v):
        def body(si_vmem, di_vmem):
            pltpu.sync_copy(x_hbm.at[si_vmem.at[0]], rows_v)
            pltpu.sync_copy(rows_v, o_hbm.at[di_vmem.at[0]])

        pltpu.emit_pipeline(
            body,
            grid=(n // win,),
            in_specs=[pl.BlockSpec((1, win), lambda i: (0, i)),
                      pl.BlockSpec((1, win), lambda i: (0, i))],
            out_specs=[],
            core_axis_name=("c", "s"),
            dimension_semantics=(pltpu.PARALLEL,),
        )(si_hbm, di_hbm)

    return k(table, si, di)


def _ffn_body(be_ref, nu_ref, ppos_ref, x_ref, w1_ref, b1_ref, w2_ref,
              b2_ref, out_ref, w1s_ref, w2s_ref):
    b = pl.program_id(0)

    @pl.when(b < nu_ref[0, 0])
    def _():
        prev = jnp.maximum(b - 1, 0)
        changed = (b == 0) | (be_ref[0, b] != be_ref[0, prev])

        @pl.when(changed)
        def _():
            w1s_ref[...] = w1_ref[0].astype(_BF)
            w2s_ref[...] = w2_ref[0].astype(_BF)

        base = b * BT
        p0 = ppos_ref[0:1, :] - base
        p1 = ppos_ref[1:2, :] - base
        slot = jax.lax.broadcasted_iota(jnp.int32, (BT, S), 0)
        oh = ((p0 == slot) | (p1 == slot)).astype(_BF)
        xv = jnp.dot(oh, x_ref[...],
                     preferred_element_type=jnp.float32).astype(_BF)
        h = jnp.dot(xv, w1s_ref[...],
                    preferred_element_type=jnp.float32) + b1_ref[0]
        h = jax.nn.gelu(h.astype(_BF))
        out_ref[...] = jnp.dot(h, w2s_ref[...],
                               preferred_element_type=jnp.float32) + b2_ref[0]


def _ffn(x_bf, ppos, block_expert, nb_used, W1, b1, W2, b2):
    w1r = W1.reshape(E, D, H)
    b1r = b1.reshape(E, 1, H)
    w2r = W2.reshape(E, H, OUTD)
    b2r = b2.reshape(E, 1, OUTD)
    grid_spec = pltpu.PrefetchScalarGridSpec(
        num_scalar_prefetch=2,
        grid=(NBCAP,),
        in_specs=[
            pl.BlockSpec((2, S), lambda b, be, nu: (0, 0)),
            pl.BlockSpec((S, D), lambda b, be, nu: (0, 0)),
            pl.BlockSpec((1, D, H), lambda b, be, nu: (be[0, b], 0, 0)),
            pl.BlockSpec((1, 1, H), lambda b, be, nu: (be[0, b], 0, 0)),
            pl.BlockSpec((1, H, OUTD), lambda b, be, nu: (be[0, b], 0, 0)),
            pl.BlockSpec((1, 1, OUTD), lambda b, be, nu: (be[0, b], 0, 0)),
        ],
        out_specs=pl.BlockSpec((BT, OUTD), lambda b, be, nu: (b, 0)),
        scratch_shapes=[pltpu.VMEM((D, H), _BF), pltpu.VMEM((H, OUTD), _BF)],
    )
    return pl.pallas_call(
        _ffn_body,
        grid_spec=grid_spec,
        out_shape=jax.ShapeDtypeStruct((PCAP, OUTD), jnp.float32),
    )(block_expert, nb_used, ppos, x_bf, w1r, b1r, w2r, b2r)


def _combine_body(yg_ref, w0_ref, w1_ref, wo_ref, bo_ref, gam_ref, bet_ref,
                  out_ref):
    comb = (w0_ref[...] * yg_ref[0:S, :] + w1_ref[...] * yg_ref[S:2 * S, :])
    z = jnp.dot(comb.astype(_BF), wo_ref[...].astype(_BF),
                preferred_element_type=jnp.float32) + bo_ref[...]
    mu = jnp.mean(z, axis=-1, keepdims=True)
    var = jnp.mean((z - mu) ** 2, axis=-1, keepdims=True)
    out_ref[...] = (z - mu) * jax.lax.rsqrt(var + 1e-5) * gam_ref[...] \
        + bet_ref[...]


def _combine(yg, w0c, w1c, Wo, bo, gamma, beta):
    return pl.pallas_call(
        _combine_body,
        out_shape=jax.ShapeDtypeStruct((S, OUTD), jnp.float32),
    )(yg, w0c, w1c, Wo, bo.reshape(1, OUTD),
      gamma.reshape(1, OUTD), beta.reshape(1, OUTD))


def kernel(x, Wg, bg, Wer, ber, W1, b1, W2, b2, Wo, bo, gamma, beta):
    x2 = x.reshape(S, D)
    x_bf2 = x2.astype(_BF)
    ppos, w, block_expert, nb_used = _route(x_bf2, Wg, bg, Wer, ber)

    ys = _ffn(x_bf2, ppos, block_expert, nb_used, W1, b1, W2, b2)
    yg = _gather_scatter(ys, ppos.reshape(2 * S),
                         jnp.arange(2 * S, dtype=jnp.int32))
    wt = w.T
    out = _combine(yg, wt[:, 0:1], wt[:, 1:2], Wo, bo, gamma, beta)
    return out.reshape(1, S, OUTD)


# speedup vs baseline: 1.0457x; 1.0457x over previous
import functools

import jax
import jax.numpy as jnp
from jax.experimental import pallas as pl
from jax.experimental.pallas import tpu as pltpu
from jax.experimental.pallas import tpu_sc as plsc

S, D, H, OUTD = 2048, 768, 3072, 768
G, EG = 4, 4
E = G * EG
BT = 256
NBCAP = (2 * S) // BT + E
PCAP = NBCAP * BT

_BF = jnp.bfloat16
_HI = jax.lax.Precision.HIGHEST


def _router_body(x_ref, wgT_ref, bg_ref, werT_ref, ber_ref,
                 ppos_ref, w_ref, be_ref, nb_ref):
    xb = x_ref[...]
    gl = jax.lax.dot_general(wgT_ref[...], xb, (((1,), (1,)), ((), ())),
                             preferred_element_type=jnp.float32) \
        + bg_ref[...]
    ridx = jax.lax.broadcasted_iota(jnp.int32, (G, S), 0)
    big = jnp.int32(G + 1)

    m = jnp.max(gl, axis=0, keepdims=True)
    egl = jnp.exp(gl - m)
    gp = egl / jnp.sum(egl, axis=0, keepdims=True)
    v1 = jnp.max(gp, axis=0, keepdims=True)
    i1 = jnp.min(jnp.where(gp == v1, ridx, big), axis=0, keepdims=True)
    gp2 = jnp.where(ridx == i1, -1.0, gp)
    v2 = jnp.max(gp2, axis=0, keepdims=True)
    i2 = jnp.min(jnp.where(gp2 == v2, ridx, big), axis=0, keepdims=True)

    ew = []
    ei = []
    for g in range(G):
        el = jax.lax.dot_general(werT_ref[g], xb, (((1,), (1,)), ((), ())),
                                 preferred_element_type=jnp.float32) \
            + ber_ref[g]
        mg = jnp.max(el, axis=0, keepdims=True)
        ei.append(jnp.min(jnp.where(el == mg, ridx, big), axis=0,
                          keepdims=True))
        ew.append(1.0 / jnp.sum(jnp.exp(el - mg), axis=0, keepdims=True))

    eids, ws = [], []
    for gsel, gwk in ((i1, v1), (i2, v2)):
        ew_sel = jnp.zeros((1, S), jnp.float32)
        ei_sel = jnp.zeros((1, S), jnp.int32)
        for g in range(G):
            hit = gsel == g
            ew_sel = jnp.where(hit, ew[g], ew_sel)
            ei_sel = jnp.where(hit, ei[g], ei_sel)
        eids.append(gsel * EG + ei_sel)
        ws.append(gwk * ew_sel)
    w_ref[...] = jnp.concatenate(ws, axis=0)

    e16 = jax.lax.broadcasted_iota(jnp.int32, (E, S), 0)
    o0 = (e16 == eids[0]).astype(jnp.float32)
    o1 = (e16 == eids[1]).astype(jnp.float32)
    t0 = jnp.sum(o0, axis=1, keepdims=True)
    t1 = jnp.sum(o1, axis=1, keepdims=True)
    cols = jax.lax.broadcasted_iota(jnp.int32, (S, S), 1)
    rows = jax.lax.broadcasted_iota(jnp.int32, (S, S), 0)
    stu = (rows < cols).astype(_BF)
    ob = jnp.concatenate([o0, o1], axis=0).astype(_BF)
    rex = jax.lax.dot_general(ob, stu, (((1,), (0,)), ((), ())),
                              preferred_element_type=jnp.float32)
    r0 = rex[0:E]
    r1 = rex[E:2 * E] + t0
    counts = t0 + t1
    pc = jnp.floor((counts + (BT - 1)) * (1.0 / BT)) * BT
    lt = (jax.lax.broadcasted_iota(jnp.int32, (E, E), 1)
          < jax.lax.broadcasted_iota(jnp.int32, (E, E), 0)).astype(jnp.float32)
    pstart = jax.lax.dot_general(lt, pc, (((1,), (0,)), ((), ())),
                                 preferred_element_type=jnp.float32,
                                 precision=_HI)
    p0 = jnp.sum(o0 * (pstart + r0), axis=0, keepdims=True)
    p1 = jnp.sum(o1 * (pstart + r1), axis=0, keepdims=True)
    ppos_ref[...] = jnp.concatenate([p0, p1], axis=0).astype(jnp.int32)
    bcols = jax.lax.broadcasted_iota(jnp.int32, (E, NBCAP), 1).astype(
        jnp.float32)
    pstart_blk = pstart * (1.0 / BT)
    be_ref[...] = (jnp.sum((pstart_blk <= bcols).astype(jnp.float32), axis=0,
                           keepdims=True) - 1.0).astype(jnp.int32)
    nb_ref[...] = (jnp.sum(pc, axis=0, keepdims=True) * (1.0 / BT)
                   ).astype(jnp.int32)


def _route(x_bf, Wg, bg, Wer, ber):
    wgT = Wg.T.astype(_BF)
    bg2 = bg.reshape(G, 1)
    werT = Wer.transpose(0, 2, 1).astype(_BF)
    ber3 = ber.reshape(G, EG, 1)
    return pl.pallas_call(
        _router_body,
        out_shape=(jax.ShapeDtypeStruct((2, S), jnp.int32),
                   jax.ShapeDtypeStruct((2, S), jnp.float32),
                   jax.ShapeDtypeStruct((1, NBCAP), jnp.int32),
                   jax.ShapeDtypeStruct((1, 1), jnp.int32)),
    )(x_bf, wgT, bg2, werT, ber3)


def _gather_scatter(table, src_idx, dst_idx):
    n = src_idx.shape[0]
    dm = table.shape[1]
    win = 128
    si = src_idx.reshape(1, n)
    di = dst_idx.reshape(1, n)
    mesh = plsc.VectorSubcoreMesh(core_axis_name="c", subcore_axis_name="s")

    @functools.partial(pl.kernel,
                       out_type=jax.ShapeDtypeStruct((n, dm), table.dtype),
                       mesh=mesh,
                       scratch_types=[pltpu.VMEM((win, dm), table.dtype)])
    def k(x_hbm, si_hbm, di_hbm, o_hbm, rows_v):
        def body(si_vmem, di_vmem):
            pltpu.sync_copy(x_hbm.at[si_vmem.at[0]], rows_v)
            pltpu.sync_copy(rows_v, o_hbm.at[di_vmem.at[0]])

        pltpu.emit_pipeline(
            body,
            grid=(n // win,),
            in_specs=[pl.BlockSpec((1, win), lambda i: (0, i)),
                      pl.BlockSpec((1, win), lambda i: (0, i))],
            out_specs=[],
            core_axis_name=("c", "s"),
            dimension_semantics=(pltpu.PARALLEL,),
        )(si_hbm, di_hbm)

    return k(table, si, di)


def _ffn_body(be_ref, nu_ref, ppos_ref, x_ref, w1_ref, b1_ref, w2_ref,
              b2_ref, out_ref):
    b = pl.program_id(0)

    @pl.when(b < nu_ref[0, 0])
    def _():
        base = b * BT
        p0 = ppos_ref[0:1, :] - base
        p1 = ppos_ref[1:2, :] - base
        slot = jax.lax.broadcasted_iota(jnp.int32, (BT, S), 0)
        oh = ((p0 == slot) | (p1 == slot)).astype(_BF)
        xv = jnp.dot(oh, x_ref[...],
                     preferred_element_type=jnp.float32).astype(_BF)
        h = jnp.dot(xv, w1_ref[0].astype(_BF),
                    preferred_element_type=jnp.float32) + b1_ref[0]
        h = jax.nn.gelu(h.astype(_BF))
        out_ref[...] = jnp.dot(h, w2_ref[0].astype(_BF),
                               preferred_element_type=jnp.float32) + b2_ref[0]


def _ffn(x_bf, ppos, block_expert, nb_used, W1, b1, W2, b2):
    w1r = W1.reshape(E, D, H)
    b1r = b1.reshape(E, 1, H)
    w2r = W2.reshape(E, H, OUTD)
    b2r = b2.reshape(E, 1, OUTD)
    grid_spec = pltpu.PrefetchScalarGridSpec(
        num_scalar_prefetch=2,
        grid=(NBCAP,),
        in_specs=[
            pl.BlockSpec((2, S), lambda b, be, nu: (0, 0)),
            pl.BlockSpec((S, D), lambda b, be, nu: (0, 0)),
            pl.BlockSpec((1, D, H), lambda b, be, nu: (be[0, b], 0, 0)),
            pl.BlockSpec((1, 1, H), lambda b, be, nu: (be[0, b], 0, 0)),
            pl.BlockSpec((1, H, OUTD), lambda b, be, nu: (be[0, b], 0, 0)),
            pl.BlockSpec((1, 1, OUTD), lambda b, be, nu: (be[0, b], 0, 0)),
        ],
        out_specs=pl.BlockSpec((BT, OUTD), lambda b, be, nu: (b, 0)),
    )
    return pl.pallas_call(
        _ffn_body,
        grid_spec=grid_spec,
        out_shape=jax.ShapeDtypeStruct((PCAP, OUTD), jnp.float32),
    )(block_expert, nb_used, ppos, x_bf, w1r, b1r, w2r, b2r)


def _combine_body(yg_ref, w0_ref, w1_ref, wo_ref, bo_ref, gam_ref, bet_ref,
                  out_ref):
    comb = (w0_ref[...] * yg_ref[0:S, :] + w1_ref[...] * yg_ref[S:2 * S, :])
    z = jnp.dot(comb.astype(_BF), wo_ref[...].astype(_BF),
                preferred_element_type=jnp.float32) + bo_ref[...]
    mu = jnp.mean(z, axis=-1, keepdims=True)
    var = jnp.mean((z - mu) ** 2, axis=-1, keepdims=True)
    out_ref[...] = (z - mu) * jax.lax.rsqrt(var + 1e-5) * gam_ref[...] \
        + bet_ref[...]


def _combine(yg, w0c, w1c, Wo, bo, gamma, beta):
    return pl.pallas_call(
        _combine_body,
        out_shape=jax.ShapeDtypeStruct((S, OUTD), jnp.float32),
    )(yg, w0c, w1c, Wo, bo.reshape(1, OUTD),
      gamma.reshape(1, OUTD), beta.reshape(1, OUTD))


def kernel(x, Wg, bg, Wer, ber, W1, b1, W2, b2, Wo, bo, gamma, beta):
    x2 = x.reshape(S, D)
    x_bf2 = x2.astype(_BF)
    ppos, w, block_expert, nb_used = _route(x_bf2, Wg, bg, Wer, ber)

    ys = _ffn(x_bf2, ppos, block_expert, nb_used, W1, b1, W2, b2)
    yg = _gather_scatter(ys, ppos.reshape(2 * S),
                         jnp.arange(2 * S, dtype=jnp.int32))
    wt = w.T
    out = _combine(yg, wt[:, 0:1], wt[:, 1:2], Wo, bo, gamma, beta)
    return out.reshape(1, S, OUTD)
